# Initial kernel scaffold; baseline (speedup 1.0000x reference)
import jax
import jax.numpy as jnp
from jax.experimental import pallas as pl


def kernel(x, edge_index, Wl1, bl1, Wr1, Wl2, bl2, Wr2):
    raise NotImplementedError("write your pallas kernel here")


# speedup vs baseline: 5.1022x; 5.1022x over previous
import functools

import jax
import jax.numpy as jnp
from jax import lax
from jax.experimental import pallas as pl
from jax.experimental.pallas import tpu as pltpu
from jax.experimental.pallas import tpu_sc as plsc

N_NODES = 10000
N_EDGES = 320000
D_IN = 128

NC = 2
NS = 16
NW = NC * NS
EPW = N_EDGES // NW
K = 80
NCHUNK = EPW // K
RPT = N_NODES // NS


def _make_sc_agg(D):
    mesh = plsc.VectorSubcoreMesh(
        core_axis_name="c", subcore_axis_name="s", num_cores=NC, num_subcores=NS
    )

    @functools.partial(
        pl.kernel,
        out_type=jax.ShapeDtypeStruct((NC, N_NODES, D), jnp.float32),
        mesh=mesh,
        compiler_params=pltpu.CompilerParams(use_tc_tiling_on_sc=False),
        scratch_types=[
            pltpu.VMEM((K,), jnp.int32),
            pltpu.VMEM((K,), jnp.int32),
            pltpu.VMEM((K, D), jnp.float32),
            pltpu.VMEM_SHARED((N_NODES, D), jnp.float32),
            pltpu.SemaphoreType.DMA,
        ],
    )
    def agg(x_hbm, src_hbm, dst_hbm, zeros_hbm, out_hbm, sidx, didx, rows, acc, sem):
        c = lax.axis_index("c")
        s = lax.axis_index("s")
        w = s * NC + c
        r0 = s * RPT
        pltpu.sync_copy(zeros_hbm.at[pl.ds(r0, RPT)], acc.at[pl.ds(r0, RPT)])
        plsc.subcore_barrier()
        base = w * EPW

        def body(i, carry):
            off = base + i * K
            pltpu.sync_copy(src_hbm.at[pl.ds(off, K)], sidx)
            pltpu.sync_copy(dst_hbm.at[pl.ds(off, K)], didx)
            pltpu.async_copy(x_hbm.at[sidx], rows, sem).wait()
            pltpu.sync_copy(rows, acc.at[didx], add=True)
            return carry

        lax.fori_loop(0, NCHUNK, body, 0)
        plsc.subcore_barrier()
        pltpu.sync_copy(acc.at[pl.ds(r0, RPT)], out_hbm.at[c, pl.ds(r0, RPT)])

    return agg


_agg144 = _make_sc_agg(144)
_agg128 = _make_sc_agg(128)

BLK = 1000


def _dense1_body(aA, aB, x_ref, wl, b, wr, h_ref, inv_ref):
    a = aA[...] + aB[...]
    sums = a[:, :D_IN]
    cnt = a[:, D_IN:D_IN + 1]
    inv = 1.0 / jnp.maximum(cnt, 1.0)
    mean = sums * inv
    h = (jnp.dot(mean, wl[...], preferred_element_type=jnp.float32)
         + b[...]
         + jnp.dot(x_ref[...], wr[...], preferred_element_type=jnp.float32))
    h_ref[...] = jnp.maximum(h, 0.0)
    inv_ref[...] = inv


def _dense1(accA, accB, x, wlT, b2d, wrT):
    grid = (N_NODES // BLK,)
    return pl.pallas_call(
        _dense1_body,
        grid=grid,
        in_specs=[
            pl.BlockSpec((BLK, 144), lambda i: (i, 0)),
            pl.BlockSpec((BLK, 144), lambda i: (i, 0)),
            pl.BlockSpec((BLK, D_IN), lambda i: (i, 0)),
            pl.BlockSpec((D_IN, D_IN), lambda i: (0, 0)),
            pl.BlockSpec((1, D_IN), lambda i: (0, 0)),
            pl.BlockSpec((D_IN, D_IN), lambda i: (0, 0)),
        ],
        out_specs=[
            pl.BlockSpec((BLK, D_IN), lambda i: (i, 0)),
            pl.BlockSpec((BLK, 1), lambda i: (i, 0)),
        ],
        out_shape=[
            jax.ShapeDtypeStruct((N_NODES, D_IN), jnp.float32),
            jax.ShapeDtypeStruct((N_NODES, 1), jnp.float32),
        ],
    )(accA, accB, x, wlT, b2d, wrT)


def _dense2_body(aA, aB, inv_ref, h_ref, wl, b, wr, o_ref):
    mean = (aA[...] + aB[...]) * inv_ref[...]
    o_ref[...] = (jnp.dot(mean, wl[...], preferred_element_type=jnp.float32)
                  + b[...]
                  + jnp.dot(h_ref[...], wr[...], preferred_element_type=jnp.float32))


def _dense2(accA, accB, inv, h, wlT, b2d, wrT):
    grid = (N_NODES // BLK,)
    return pl.pallas_call(
        _dense2_body,
        grid=grid,
        in_specs=[
            pl.BlockSpec((BLK, D_IN), lambda i: (i, 0)),
            pl.BlockSpec((BLK, D_IN), lambda i: (i, 0)),
            pl.BlockSpec((BLK, 1), lambda i: (i, 0)),
            pl.BlockSpec((BLK, D_IN), lambda i: (i, 0)),
            pl.BlockSpec((D_IN, D_IN), lambda i: (0, 0)),
            pl.BlockSpec((1, D_IN), lambda i: (0, 0)),
            pl.BlockSpec((D_IN, D_IN), lambda i: (0, 0)),
        ],
        out_specs=pl.BlockSpec((BLK, D_IN), lambda i: (i, 0)),
        out_shape=jax.ShapeDtypeStruct((N_NODES, D_IN), jnp.float32),
    )(accA, accB, inv, h, wlT, b2d, wrT)


def kernel(x, edge_index, Wl1, bl1, Wr1, Wl2, bl2, Wr2):
    src = edge_index[0].astype(jnp.int32)
    dst = edge_index[1].astype(jnp.int32)
    x_aug = jnp.concatenate(
        [x, jnp.ones((N_NODES, 1), jnp.float32), jnp.zeros((N_NODES, 15), jnp.float32)],
        axis=1,
    )
    z144 = jnp.zeros((N_NODES, 144), jnp.float32)
    z128 = jnp.zeros((N_NODES, D_IN), jnp.float32)

    acc1 = _agg144(x_aug, src, dst, z144)
    h, inv = _dense1(acc1[0], acc1[1], x, Wl1.T, bl1[None, :], Wr1.T)
    acc2 = _agg128(h, src, dst, z128)
    out = _dense2(acc2[0], acc2[1], inv, h, Wl2.T, bl2[None, :], Wr2.T)
    return out


# speedup vs baseline: 9.1265x; 1.7887x over previous
import functools

import jax
import jax.numpy as jnp
from jax import lax
from jax.experimental import pallas as pl
from jax.experimental.pallas import tpu as pltpu
from jax.experimental.pallas import tpu_sc as plsc

N_NODES = 10000
N_EDGES = 320000
D_IN = 128

NC = 2
NS = 16
EPW = N_EDGES // NS
K = 80
NCHUNK = EPW // K
RPT = N_NODES // NS


def _make_sc_agg(CW):
    mesh = plsc.VectorSubcoreMesh(
        core_axis_name="c", subcore_axis_name="s", num_cores=NC, num_subcores=NS
    )

    @functools.partial(
        pl.kernel,
        out_type=jax.ShapeDtypeStruct((N_NODES, NC * CW), jnp.float32),
        mesh=mesh,
        compiler_params=pltpu.CompilerParams(use_tc_tiling_on_sc=False),
        scratch_types=[
            pltpu.VMEM((NCHUNK, K), jnp.int32),
            pltpu.VMEM((NCHUNK, K), jnp.int32),
            pltpu.VMEM((K, CW), jnp.float32),
            pltpu.VMEM((K, CW), jnp.float32),
            pltpu.VMEM_SHARED((N_NODES, CW), jnp.float32),
            pltpu.SemaphoreType.DMA,
            pltpu.SemaphoreType.DMA,
        ],
    )
    def agg(tbl_hbm, src_hbm, dst_hbm, zeros_hbm, out_hbm,
            sidx, didx, rows0, rows1, acc, sem0, sem1):
        c = lax.axis_index("c")
        s = lax.axis_index("s")
        r0 = s * RPT
        pltpu.sync_copy(src_hbm.at[s], sidx)
        pltpu.sync_copy(dst_hbm.at[s], didx)
        pltpu.sync_copy(zeros_hbm.at[pl.ds(r0, RPT)], acc.at[pl.ds(r0, RPT)])
        plsc.subcore_barrier()

        def gather(i, rows, sem):
            pltpu.async_copy(tbl_hbm.at[c].at[sidx.at[i]], rows, sem)

        def gwait(i, rows, sem):
            pltpu.make_async_copy(tbl_hbm.at[c].at[sidx.at[i]], rows, sem).wait()

        def scatter(i, rows):
            pltpu.sync_copy(rows, acc.at[didx.at[i]], add=True)

        gather(0, rows0, sem0)

        def body(j, carry):
            a = 2 * j
            gather(a + 1, rows1, sem1)
            gwait(a, rows0, sem0)
            scatter(a, rows0)
            gather(a + 2, rows0, sem0)
            gwait(a + 1, rows1, sem1)
            scatter(a + 1, rows1)
            return carry

        lax.fori_loop(0, NCHUNK // 2 - 1, body, 0)
        gather(NCHUNK - 1, rows1, sem1)
        gwait(NCHUNK - 2, rows0, sem0)
        scatter(NCHUNK - 2, rows0)
        gwait(NCHUNK - 1, rows1, sem1)
        scatter(NCHUNK - 1, rows1)

        plsc.subcore_barrier()
        pltpu.sync_copy(acc.at[pl.ds(r0, RPT)],
                        out_hbm.at[pl.ds(r0, RPT), pl.ds(c * CW, CW)])

    return agg


_agg80 = _make_sc_agg(80)
_agg64 = _make_sc_agg(64)

BLK = 1000


def _dense1_body(a_ref, x_ref, wl, b, wr, h_ref, inv_ref):
    a = a_ref[...]
    sums = a[:, :D_IN]
    cnt = a[:, D_IN:D_IN + 1]
    inv = 1.0 / jnp.maximum(cnt, 1.0)
    mean = sums * inv
    h = (jnp.dot(mean, wl[...], preferred_element_type=jnp.float32)
         + b[...]
         + jnp.dot(x_ref[...], wr[...], preferred_element_type=jnp.float32))
    h = jnp.maximum(h, 0.0)
    h_ref[0] = h[:, :64]
    h_ref[1] = h[:, 64:]
    inv_ref[...] = inv


def _dense1(acc, x, wlT, b2d, wrT):
    grid = (N_NODES // BLK,)
    return pl.pallas_call(
        _dense1_body,
        grid=grid,
        in_specs=[
            pl.BlockSpec((BLK, 160), lambda i: (i, 0)),
            pl.BlockSpec((BLK, D_IN), lambda i: (i, 0)),
            pl.BlockSpec((D_IN, D_IN), lambda i: (0, 0)),
            pl.BlockSpec((1, D_IN), lambda i: (0, 0)),
            pl.BlockSpec((D_IN, D_IN), lambda i: (0, 0)),
        ],
        out_specs=[
            pl.BlockSpec((2, BLK, 64), lambda i: (0, i, 0)),
            pl.BlockSpec((BLK, 1), lambda i: (i, 0)),
        ],
        out_shape=[
            jax.ShapeDtypeStruct((2, N_NODES, 64), jnp.float32),
            jax.ShapeDtypeStruct((N_NODES, 1), jnp.float32),
        ],
    )(acc, x, wlT, b2d, wrT)


def _dense2_body(a_ref, inv_ref, h_ref, wl, b, wr, o_ref):
    mean = a_ref[...] * inv_ref[...]
    h = jnp.concatenate([h_ref[0], h_ref[1]], axis=1)
    o_ref[...] = (jnp.dot(mean, wl[...], preferred_element_type=jnp.float32)
                  + b[...]
                  + jnp.dot(h, wr[...], preferred_element_type=jnp.float32))


def _dense2(acc, inv, h_tbl, wlT, b2d, wrT):
    grid = (N_NODES // BLK,)
    return pl.pallas_call(
        _dense2_body,
        grid=grid,
        in_specs=[
            pl.BlockSpec((BLK, D_IN), lambda i: (i, 0)),
            pl.BlockSpec((BLK, 1), lambda i: (i, 0)),
            pl.BlockSpec((2, BLK, 64), lambda i: (0, i, 0)),
            pl.BlockSpec((D_IN, D_IN), lambda i: (0, 0)),
            pl.BlockSpec((1, D_IN), lambda i: (0, 0)),
            pl.BlockSpec((D_IN, D_IN), lambda i: (0, 0)),
        ],
        out_specs=pl.BlockSpec((BLK, D_IN), lambda i: (i, 0)),
        out_shape=jax.ShapeDtypeStruct((N_NODES, D_IN), jnp.float32),
    )(acc, inv, h_tbl, wlT, b2d, wrT)


def kernel(x, edge_index, Wl1, bl1, Wr1, Wl2, bl2, Wr2):
    src = edge_index[0].astype(jnp.int32).reshape(NS, NCHUNK, K)
    dst = edge_index[1].astype(jnp.int32).reshape(NS, NCHUNK, K)
    tbl1 = jnp.stack([
        x[:, :80],
        jnp.concatenate(
            [x[:, 80:], jnp.ones((N_NODES, 1), jnp.float32),
             jnp.zeros((N_NODES, 31), jnp.float32)], axis=1),
    ])
    z80 = jnp.zeros((N_NODES, 80), jnp.float32)
    z64 = jnp.zeros((N_NODES, 64), jnp.float32)

    acc1 = _agg80(tbl1, src, dst, z80)
    h_tbl, inv = _dense1(acc1, x, Wl1.T, bl1[None, :], Wr1.T)
    acc2 = _agg64(h_tbl, src, dst, z64)
    out = _dense2(acc2, inv, h_tbl, Wl2.T, bl2[None, :], Wr2.T)
    return out


# speedup vs baseline: 9.7403x; 1.0673x over previous
import functools

import jax
import jax.numpy as jnp
from jax import lax
from jax.experimental import pallas as pl
from jax.experimental.pallas import tpu as pltpu
from jax.experimental.pallas import tpu_sc as plsc

N_NODES = 10000
N_EDGES = 320000
D_IN = 128

NC = 2
NS = 16
CW = 64
K = 128
EPW = N_EDGES // NS
NCHUNK = -(-EPW // K)
EPW_PAD = NCHUNK * K
N_PAD = N_NODES + 16
RPT = N_NODES // NS


def _make_sc_agg(with_count):
    mesh = plsc.VectorSubcoreMesh(
        core_axis_name="c", subcore_axis_name="s", num_cores=NC, num_subcores=NS
    )
    out_type = [jax.ShapeDtypeStruct((N_NODES, NC * CW), jnp.float32)]
    scratch = [
        pltpu.VMEM((NCHUNK, K), jnp.int32),
        pltpu.VMEM((NCHUNK, K), jnp.int32),
        pltpu.VMEM((K, CW), jnp.float32),
        pltpu.VMEM((K, CW), jnp.float32),
        pltpu.VMEM_SHARED((N_PAD, CW), jnp.float32),
        pltpu.SemaphoreType.DMA,
        pltpu.SemaphoreType.DMA,
    ]
    if with_count:
        out_type.append(jax.ShapeDtypeStruct((N_NODES,), jnp.float32))
        scratch.append(pltpu.VMEM((K,), jnp.float32))
        scratch.append(pltpu.VMEM_SHARED((N_PAD,), jnp.float32))

    @functools.partial(
        pl.kernel,
        out_type=out_type,
        mesh=mesh,
        compiler_params=pltpu.CompilerParams(use_tc_tiling_on_sc=False),
        scratch_types=scratch,
    )
    def agg(tbl_hbm, src_hbm, dst_hbm, zeros_hbm, *rest):
        if with_count:
            (zcnt_hbm, out_hbm, cnt_hbm,
             sidx, didx, rows0, rows1, acc, sem0, sem1, ones, cnt) = rest
        else:
            (out_hbm, sidx, didx, rows0, rows1, acc, sem0, sem1) = rest
        c = lax.axis_index("c")
        s = lax.axis_index("s")
        r0 = s * RPT
        pltpu.sync_copy(src_hbm.at[s], sidx)
        pltpu.sync_copy(dst_hbm.at[s], didx)
        pltpu.sync_copy(zeros_hbm.at[pl.ds(r0, RPT)], acc.at[pl.ds(r0, RPT)])
        if with_count:
            @pl.when(s == 0)
            def _():
                pltpu.sync_copy(zcnt_hbm, cnt.at[pl.ds(0, N_NODES)])
            for v in range(K // 16):
                ones[pl.ds(v * 16, 16)] = jnp.ones((16,), jnp.float32)
        plsc.subcore_barrier()

        def gather(i, rows, sem):
            pltpu.async_copy(tbl_hbm.at[c].at[sidx.at[i]], rows, sem)

        def gwait(i, rows, sem):
            pltpu.make_async_copy(tbl_hbm.at[c].at[sidx.at[i]], rows, sem).wait()

        def scatter(i, rows):
            pltpu.sync_copy(rows, acc.at[didx.at[i]], add=True)
            if with_count:
                pltpu.sync_copy(ones, cnt.at[didx.at[i]], add=True)

        gather(0, rows0, sem0)

        def body(j, carry):
            a = 2 * j
            gather(a + 1, rows1, sem1)
            gwait(a, rows0, sem0)
            scatter(a, rows0)
            gather(a + 2, rows0, sem0)
            gwait(a + 1, rows1, sem1)
            scatter(a + 1, rows1)
            return carry

        lax.fori_loop(0, (NCHUNK - 1) // 2, body, 0)
        gwait(NCHUNK - 1, rows0, sem0)
        scatter(NCHUNK - 1, rows0)

        plsc.subcore_barrier()
        pltpu.sync_copy(acc.at[pl.ds(r0, RPT)],
                        out_hbm.at[pl.ds(r0, RPT), pl.ds(c * CW, CW)])
        if with_count:
            @pl.when((c == 0) & (s == 0))
            def _():
                pltpu.sync_copy(cnt.at[pl.ds(0, N_NODES)], cnt_hbm)

    return agg


_agg_l1 = _make_sc_agg(True)
_agg_l2 = _make_sc_agg(False)

BLK = 1000


def _dense1_body(a_ref, cnt_ref, x_ref, wl, b, wr, h_ref, inv_ref):
    inv = 1.0 / jnp.maximum(cnt_ref[...], 1.0)
    mean = a_ref[...] * inv
    h = (jnp.dot(mean, wl[...], preferred_element_type=jnp.float32)
         + b[...]
         + jnp.dot(x_ref[...], wr[...], preferred_element_type=jnp.float32))
    h = jnp.maximum(h, 0.0)
    h_ref[0] = h[:, :CW]
    h_ref[1] = h[:, CW:]
    inv_ref[...] = inv


def _dense1(acc, cnt2d, x, wlT, b2d, wrT):
    grid = (N_NODES // BLK,)
    return pl.pallas_call(
        _dense1_body,
        grid=grid,
        in_specs=[
            pl.BlockSpec((BLK, D_IN), lambda i: (i, 0)),
            pl.BlockSpec((BLK, 1), lambda i: (i, 0)),
            pl.BlockSpec((BLK, D_IN), lambda i: (i, 0)),
            pl.BlockSpec((D_IN, D_IN), lambda i: (0, 0)),
            pl.BlockSpec((1, D_IN), lambda i: (0, 0)),
            pl.BlockSpec((D_IN, D_IN), lambda i: (0, 0)),
        ],
        out_specs=[
            pl.BlockSpec((2, BLK, CW), lambda i: (0, i, 0)),
            pl.BlockSpec((BLK, 1), lambda i: (i, 0)),
        ],
        out_shape=[
            jax.ShapeDtypeStruct((2, N_NODES, CW), jnp.float32),
            jax.ShapeDtypeStruct((N_NODES, 1), jnp.float32),
        ],
    )(acc, cnt2d, x, wlT, b2d, wrT)


def _dense2_body(a_ref, inv_ref, h_ref, wl, b, wr, o_ref):
    mean = a_ref[...] * inv_ref[...]
    h = jnp.concatenate([h_ref[0], h_ref[1]], axis=1)
    o_ref[...] = (jnp.dot(mean, wl[...], preferred_element_type=jnp.float32)
                  + b[...]
                  + jnp.dot(h, wr[...], preferred_element_type=jnp.float32))


def _dense2(acc, inv, h_tbl, wlT, b2d, wrT):
    grid = (N_NODES // BLK,)
    return pl.pallas_call(
        _dense2_body,
        grid=grid,
        in_specs=[
            pl.BlockSpec((BLK, D_IN), lambda i: (i, 0)),
            pl.BlockSpec((BLK, 1), lambda i: (i, 0)),
            pl.BlockSpec((2, BLK, CW), lambda i: (0, i, 0)),
            pl.BlockSpec((D_IN, D_IN), lambda i: (0, 0)),
            pl.BlockSpec((1, D_IN), lambda i: (0, 0)),
            pl.BlockSpec((D_IN, D_IN), lambda i: (0, 0)),
        ],
        out_specs=pl.BlockSpec((BLK, D_IN), lambda i: (i, 0)),
        out_shape=jax.ShapeDtypeStruct((N_NODES, D_IN), jnp.float32),
    )(acc, inv, h_tbl, wlT, b2d, wrT)


def _pad_idx(v, fill):
    v = v.reshape(NS, EPW)
    v = jnp.pad(v, ((0, 0), (0, EPW_PAD - EPW)), constant_values=fill)
    return v.reshape(NS, NCHUNK, K)


def kernel(x, edge_index, Wl1, bl1, Wr1, Wl2, bl2, Wr2):
    src = _pad_idx(edge_index[0].astype(jnp.int32), 0)
    dst = _pad_idx(edge_index[1].astype(jnp.int32), N_NODES)
    tbl1 = jnp.stack([x[:, :CW], x[:, CW:]])
    z64 = jnp.zeros((N_NODES, CW), jnp.float32)
    zcnt = jnp.zeros((N_NODES,), jnp.float32)

    acc1, cnt = _agg_l1(tbl1, src, dst, z64, zcnt)
    h_tbl, inv = _dense1(acc1, cnt[:, None], x, Wl1.T, bl1[None, :], Wr1.T)
    acc2, = _agg_l2(h_tbl, src, dst, z64)
    out = _dense2(acc2, inv, h_tbl, Wl2.T, bl2[None, :], Wr2.T)
    return out
